# baseline (speedup 1.0000x reference)
import functools

import jax
import jax.numpy as jnp
import numpy as np
from jax import lax
from jax.experimental import pallas as pl
from jax.experimental.pallas import tpu as pltpu
from jax.experimental.pallas import tpu_sc as plsc

N = 10000
E = 320000
D = 128

NC = 2
NS = 16
LANES = 16
NW = NC * NS

CHUNK = 128
CPT = 80
EP = NW * CPT * CHUNK
NPAD = 10240
RPT = NPAD // NS
PAD_IDX = N + 16

MC = 64
MCPT = EP // (NW * MC)
BLK = 8
NBLK = MCPT // BLK

_sc_mesh = plsc.VectorSubcoreMesh(core_axis_name="c", subcore_axis_name="s")


def _deg_body(col_hbm, ew_hbm, degp_hbm, col_v, ew_v, zb_v, acc_sh):
    c = lax.axis_index("c")
    s = lax.axis_index("s")
    w = c * NS + s

    pltpu.sync_copy(col_hbm.at[pl.ds(w * CPT, CPT)], col_v)
    pltpu.sync_copy(ew_hbm.at[pl.ds(w * CPT, CPT)], ew_v)

    def zero(i, carry):
        zb_v[pl.ds(i * LANES, LANES)] = jnp.zeros((LANES,), jnp.float32)
        return carry

    lax.fori_loop(0, RPT // LANES, zero, 0)
    pltpu.sync_copy(zb_v, acc_sh.at[pl.ds(s * RPT, RPT)])
    plsc.subcore_barrier()

    def body(j, carry):
        pltpu.sync_copy(ew_v.at[j], acc_sh.at[col_v.at[j]], add=True)
        return carry

    lax.fori_loop(0, CPT, body, 0)
    plsc.subcore_barrier()
    pltpu.sync_copy(
        acc_sh.at[pl.ds(s * RPT, RPT)],
        degp_hbm.at[c, pl.ds(s * RPT, RPT)],
    )


_sc_params = pltpu.CompilerParams(needs_layout_passes=False)

_deg_call = pl.kernel(
    _deg_body,
    out_type=jax.ShapeDtypeStruct((NC, NPAD), jnp.float32),
    mesh=_sc_mesh,
    compiler_params=_sc_params,
    scratch_types=[
        pltpu.VMEM((CPT, CHUNK), jnp.int32),
        pltpu.VMEM((CPT, CHUNK), jnp.float32),
        pltpu.VMEM((RPT,), jnp.float32),
        pltpu.VMEM_SHARED((NPAD,), jnp.float32),
    ],
)


def _gw_body(row_hbm, ew_hbm, dis_hbm, gw_hbm, row_v, ew_v, g_v, dis_v):
    c = lax.axis_index("c")
    s = lax.axis_index("s")
    w = c * NS + s

    pltpu.sync_copy(row_hbm.at[pl.ds(w * MCPT, MCPT)], row_v)
    pltpu.sync_copy(ew_hbm.at[pl.ds(w * MCPT, MCPT)], ew_v)
    pltpu.sync_copy(dis_hbm, dis_v)

    def body(jj, carry):
        for gi in range(MC // LANES):
            sl = pl.ds(gi * LANES, LANES)
            d16 = plsc.load_gather(dis_v, [row_v[jj, sl]])
            g_v[jj, sl] = d16 * ew_v[jj, sl]
        return carry

    lax.fori_loop(0, MCPT, body, 0)
    pltpu.sync_copy(g_v, gw_hbm.at[pl.ds(w * MCPT, MCPT)])


_gw_call = pl.kernel(
    _gw_body,
    out_type=jax.ShapeDtypeStruct((EP // MC, MC), jnp.float32),
    mesh=_sc_mesh,
    compiler_params=_sc_params,
    scratch_types=[
        pltpu.VMEM((MCPT, MC), jnp.int32),
        pltpu.VMEM((MCPT, MC), jnp.float32),
        pltpu.VMEM((MCPT, MC), jnp.float32),
        pltpu.VMEM((NPAD,), jnp.float32),
    ],
)


def _msg_body(row_hbm, col_hbm, gw_hbm, h_hbm, accp_hbm,
              row_v, col_v, gw_v, ga, gb, sa, sb, acc_sh,
              g0, g1, s0, s1, stg):
    c = lax.axis_index("c")
    s = lax.axis_index("s")
    w = c * NS + s
    tbase = w * MCPT

    gbufs = [ga, gb]
    sbufs = [sa, sb]
    gsems = [g0, g1]
    ssems = [s0, s1]

    def zrow(i, carry):
        for v in range(D // LANES):
            sa[i, pl.ds(v * LANES, LANES)] = jnp.zeros(
                (LANES,), jnp.float32)
        return carry

    lax.fori_loop(0, MC, zrow, 0)

    def zcopy(kz, carry):
        pltpu.sync_copy(sa, acc_sh.at[pl.ds(s * RPT + kz * MC, MC)])
        return carry

    lax.fori_loop(0, RPT // MC, zcopy, 0)
    plsc.subcore_barrier()

    def scale_chunk(gbuf, sbuf, p):
        def grp(gi, carry):
            lanes = lax.iota(jnp.int32, LANES)
            e16 = gi * LANES + lanes
            g16 = gw_v[p, pl.ds(gi * LANES, LANES)]
            for f in range(D):
                fo = (lanes + f) & (D - 1)
                vals = plsc.load_gather(gbuf, [e16, fo])
                plsc.store_scatter(sbuf, [e16, fo], vals * g16)
            return carry

        lax.fori_loop(0, MC // LANES, grp, 0)

    pltpu.sync_copy(row_hbm.at[pl.ds(tbase, BLK)], row_v.at[0])
    pltpu.sync_copy(col_hbm.at[pl.ds(tbase, BLK)], col_v.at[0])
    pltpu.sync_copy(gw_hbm.at[pl.ds(tbase, BLK)], gw_v)
    pltpu.async_copy(h_hbm.at[row_v.at[0, 0]], ga, g0)

    def block(bb, carry):
        par = bb & 1
        base = tbase + bb * BLK

        @pl.when(bb > 0)
        def _():
            pltpu.sync_copy(col_hbm.at[pl.ds(base, BLK)], col_v.at[par])
            pltpu.sync_copy(gw_hbm.at[pl.ds(base, BLK)], gw_v)

        @pl.when(bb < NBLK - 1)
        def _():
            pltpu.async_copy(row_hbm.at[pl.ds(base + BLK, BLK)],
                             row_v.at[1 - par], stg)

        def duo(k, carry1):
            for u in range(2):
                p = 2 * k + u
                pltpu.make_async_copy(h_hbm.at[row_v.at[par, p]],
                                      gbufs[u], gsems[u]).wait()
                if u == 0:
                    pltpu.async_copy(h_hbm.at[row_v.at[par, p + 1]],
                                     gbufs[1], gsems[1])
                else:
                    @pl.when(k < BLK // 2 - 1)
                    def _():
                        pltpu.async_copy(h_hbm.at[row_v.at[par, p + 1]],
                                         gbufs[0], gsems[0])

                    @pl.when((k == BLK // 2 - 1) & (bb < NBLK - 1))
                    def _():
                        pltpu.make_async_copy(
                            row_hbm.at[pl.ds(base + BLK, BLK)],
                            row_v.at[1 - par], stg).wait()
                        pltpu.async_copy(h_hbm.at[row_v.at[1 - par, 0]],
                                         gbufs[0], gsems[0])

                @pl.when((bb > 0) | (k > 0))
                def _():
                    pltpu.make_async_copy(
                        sbufs[u], acc_sh.at[col_v.at[par, p]],
                        ssems[u]).wait()

                scale_chunk(gbufs[u], sbufs[u], p)
                pltpu.async_copy(sbufs[u], acc_sh.at[col_v.at[par, p]],
                                 ssems[u], add=True)
            return carry1

        lax.fori_loop(0, BLK // 2, duo, 0)
        return carry

    lax.fori_loop(0, NBLK, block, 0)

    for u in range(2):
        pltpu.make_async_copy(sbufs[u], acc_sh.at[col_v.at[0, 0]],
                              ssems[u]).wait()

    plsc.subcore_barrier()
    pltpu.sync_copy(
        acc_sh.at[pl.ds(s * RPT, RPT)],
        accp_hbm.at[c, pl.ds(s * RPT, RPT)],
    )


_msg_call = pl.kernel(
    _msg_body,
    out_type=jax.ShapeDtypeStruct((NC, NPAD, D), jnp.float32),
    mesh=_sc_mesh,
    compiler_params=_sc_params,
    scratch_types=[
        pltpu.VMEM((2, BLK, MC), jnp.int32),
        pltpu.VMEM((2, BLK, MC), jnp.int32),
        pltpu.VMEM((BLK, MC), jnp.float32),
        pltpu.VMEM((MC, D), jnp.float32),
        pltpu.VMEM((MC, D), jnp.float32),
        pltpu.VMEM((MC, D), jnp.float32),
        pltpu.VMEM((MC, D), jnp.float32),
        pltpu.VMEM_SHARED((NPAD, D), jnp.float32),
        pltpu.SemaphoreType.DMA,
        pltpu.SemaphoreType.DMA,
        pltpu.SemaphoreType.DMA,
        pltpu.SemaphoreType.DMA,
        pltpu.SemaphoreType.DMA,
    ],
)


MM_BLK = 1280


def _mm_body(x_ref, w_ref, o_ref):
    o_ref[...] = jnp.dot(x_ref[...], w_ref[...],
                         preferred_element_type=jnp.float32)


_mm_call = pl.pallas_call(
    _mm_body,
    grid=(NPAD // MM_BLK,),
    in_specs=[
        pl.BlockSpec((MM_BLK, D), lambda i: (i, 0)),
        pl.BlockSpec((D, D), lambda i: (0, 0)),
    ],
    out_specs=pl.BlockSpec((MM_BLK, D), lambda i: (i, 0)),
    out_shape=jax.ShapeDtypeStruct((NPAD, D), jnp.float32),
)


def _dis_body(degp_ref, dis_ref):
    deg = degp_ref[0, :] + degp_ref[1, :]
    safe = jnp.where(deg > 0, deg, 1.0)
    dis_ref[...] = jnp.where(deg > 0, lax.rsqrt(safe), 0.0)


_dis_call = pl.pallas_call(
    _dis_body,
    out_shape=jax.ShapeDtypeStruct((NPAD,), jnp.float32),
)


FIN_BLK = 1280


def _fin_body(accp_ref, dis_ref, b_ref, o_ref):
    acc = accp_ref[0] + accp_ref[1]
    o = acc * dis_ref[...] + b_ref[...]
    m = jnp.max(o, axis=1, keepdims=True)
    e = jnp.exp(o - m)
    o_ref[...] = e / jnp.sum(e, axis=1, keepdims=True)


_fin_call = pl.pallas_call(
    _fin_body,
    grid=(NPAD // FIN_BLK,),
    in_specs=[
        pl.BlockSpec((NC, FIN_BLK, D), lambda i: (0, i, 0)),
        pl.BlockSpec((FIN_BLK, 1), lambda i: (i, 0)),
        pl.BlockSpec((1, D), lambda i: (0, 0)),
    ],
    out_specs=pl.BlockSpec((FIN_BLK, D), lambda i: (i, 0)),
    out_shape=jax.ShapeDtypeStruct((NPAD, D), jnp.float32),
)


def kernel(x, edge_index, edge_weight, W, b):
    row = edge_index[0].astype(jnp.int32)
    col = edge_index[1].astype(jnp.int32)
    pad = EP - E
    rowp = jnp.concatenate([row, jnp.full((pad,), PAD_IDX, jnp.int32)])
    colp = jnp.concatenate([col, jnp.full((pad,), PAD_IDX, jnp.int32)])
    ewp = jnp.concatenate(
        [edge_weight.astype(jnp.float32), jnp.zeros((pad,), jnp.float32)])
    xp = jnp.concatenate(
        [x.astype(jnp.float32), jnp.zeros((NPAD - N, D), jnp.float32)])

    degp = _deg_call(colp.reshape(EP // CHUNK, CHUNK),
                     ewp.reshape(EP // CHUNK, CHUNK))
    h = _mm_call(xp, W.astype(jnp.float32))
    dis = _dis_call(degp)
    row64 = rowp.reshape(EP // MC, MC)
    gw = _gw_call(row64, ewp.reshape(EP // MC, MC), dis)
    accp = _msg_call(row64, colp.reshape(EP // MC, MC), gw, h)
    out = _fin_call(accp, dis.reshape(NPAD, 1),
                    b.astype(jnp.float32).reshape(1, D))
    return out[:N]


# speedup vs baseline: 1.0845x; 1.0845x over previous
import functools

import jax
import jax.numpy as jnp
import numpy as np
from jax import lax
from jax.experimental import pallas as pl
from jax.experimental.pallas import tpu as pltpu
from jax.experimental.pallas import tpu_sc as plsc

N = 10000
E = 320000
D = 128

NC = 2
NS = 16
LANES = 16
NW = NC * NS

CHUNK = 128
CPT = 80
EP = NW * CPT * CHUNK
NPAD = 10240
RPT = NPAD // NS
PAD_IDX = N + 16

MC = 64
MCPT = EP // (NW * MC)
BLK = 8
NBLK = MCPT // BLK

_sc_mesh = plsc.VectorSubcoreMesh(core_axis_name="c", subcore_axis_name="s")


def _deg_body(col_hbm, ew_hbm, degp_hbm, col_v, ew_v, zb_v, acc_sh):
    c = lax.axis_index("c")
    s = lax.axis_index("s")
    w = c * NS + s

    pltpu.sync_copy(col_hbm.at[pl.ds(w * CPT, CPT)], col_v)
    pltpu.sync_copy(ew_hbm.at[pl.ds(w * CPT, CPT)], ew_v)

    def zero(i, carry):
        zb_v[pl.ds(i * LANES, LANES)] = jnp.zeros((LANES,), jnp.float32)
        return carry

    lax.fori_loop(0, RPT // LANES, zero, 0)
    pltpu.sync_copy(zb_v, acc_sh.at[pl.ds(s * RPT, RPT)])
    plsc.subcore_barrier()

    def body(j, carry):
        pltpu.sync_copy(ew_v.at[j], acc_sh.at[col_v.at[j]], add=True)
        return carry

    lax.fori_loop(0, CPT, body, 0)
    plsc.subcore_barrier()
    pltpu.sync_copy(
        acc_sh.at[pl.ds(s * RPT, RPT)],
        degp_hbm.at[c, pl.ds(s * RPT, RPT)],
    )


_sc_params = pltpu.CompilerParams(needs_layout_passes=False)

_deg_call = pl.kernel(
    _deg_body,
    out_type=jax.ShapeDtypeStruct((NC, NPAD), jnp.float32),
    mesh=_sc_mesh,
    compiler_params=_sc_params,
    scratch_types=[
        pltpu.VMEM((CPT, CHUNK), jnp.int32),
        pltpu.VMEM((CPT, CHUNK), jnp.float32),
        pltpu.VMEM((RPT,), jnp.float32),
        pltpu.VMEM_SHARED((NPAD,), jnp.float32),
    ],
)


def _gw_body(row_hbm, ew_hbm, dis_hbm, gw_hbm, row_v, ew_v, g_v, dis_v):
    c = lax.axis_index("c")
    s = lax.axis_index("s")
    w = c * NS + s

    pltpu.sync_copy(row_hbm.at[pl.ds(w * MCPT, MCPT)], row_v)
    pltpu.sync_copy(ew_hbm.at[pl.ds(w * MCPT, MCPT)], ew_v)
    pltpu.sync_copy(dis_hbm, dis_v)

    def body(jj, carry):
        for gi in range(MC // LANES):
            sl = pl.ds(gi * LANES, LANES)
            d16 = plsc.load_gather(dis_v, [row_v[jj, sl]])
            g_v[jj, sl] = d16 * ew_v[jj, sl]
        return carry

    lax.fori_loop(0, MCPT, body, 0)
    pltpu.sync_copy(g_v, gw_hbm.at[pl.ds(w * MCPT, MCPT)])


_gw_call = pl.kernel(
    _gw_body,
    out_type=jax.ShapeDtypeStruct((EP // MC, MC), jnp.float32),
    mesh=_sc_mesh,
    compiler_params=_sc_params,
    scratch_types=[
        pltpu.VMEM((MCPT, MC), jnp.int32),
        pltpu.VMEM((MCPT, MC), jnp.float32),
        pltpu.VMEM((MCPT, MC), jnp.float32),
        pltpu.VMEM((NPAD,), jnp.float32),
    ],
)


def _msg_body(row_hbm, col_hbm, gw_hbm, h_hbm, accp_hbm,
              row_v, col_v, gw_v, ga, gb, sa, sb, acc_sh,
              g0, g1, s0, s1, stg):
    c = lax.axis_index("c")
    s = lax.axis_index("s")
    w = c * NS + s
    tbase = w * MCPT

    gbufs = [ga, gb]
    sbufs = [sa, sb]
    gsems = [g0, g1]
    ssems = [s0, s1]

    def zrow(i, carry):
        for v in range(D // LANES):
            sa[i, pl.ds(v * LANES, LANES)] = jnp.zeros(
                (LANES,), jnp.float32)
        return carry

    lax.fori_loop(0, MC, zrow, 0)

    def zcopy(kz, carry):
        pltpu.sync_copy(sa, acc_sh.at[pl.ds(s * RPT + kz * MC, MC)])
        return carry

    lax.fori_loop(0, RPT // MC, zcopy, 0)
    plsc.subcore_barrier()

    def scale_chunk(gbuf, sbuf, p):
        def grp(gi, carry):
            g16 = gw_v[p, pl.ds(gi * LANES, LANES)]
            for j in range(LANES):
                e = gi * LANES + j
                g = jnp.broadcast_to(g16[j], (LANES,))
                for v in range(D // LANES):
                    sl = pl.ds(v * LANES, LANES)
                    sbuf[e, sl] = gbuf[e, sl] * g
            return carry

        lax.fori_loop(0, MC // LANES, grp, 0)

    pltpu.sync_copy(row_hbm.at[pl.ds(tbase, BLK)], row_v.at[0])
    pltpu.sync_copy(col_hbm.at[pl.ds(tbase, BLK)], col_v.at[0])
    pltpu.sync_copy(gw_hbm.at[pl.ds(tbase, BLK)], gw_v)
    pltpu.async_copy(h_hbm.at[row_v.at[0, 0]], ga, g0)

    def block(bb, carry):
        par = bb & 1
        base = tbase + bb * BLK

        @pl.when(bb > 0)
        def _():
            pltpu.sync_copy(col_hbm.at[pl.ds(base, BLK)], col_v.at[par])
            pltpu.sync_copy(gw_hbm.at[pl.ds(base, BLK)], gw_v)

        @pl.when(bb < NBLK - 1)
        def _():
            pltpu.async_copy(row_hbm.at[pl.ds(base + BLK, BLK)],
                             row_v.at[1 - par], stg)

        def duo(k, carry1):
            for u in range(2):
                p = 2 * k + u
                pltpu.make_async_copy(h_hbm.at[row_v.at[par, p]],
                                      gbufs[u], gsems[u]).wait()
                if u == 0:
                    pltpu.async_copy(h_hbm.at[row_v.at[par, p + 1]],
                                     gbufs[1], gsems[1])
                else:
                    @pl.when(k < BLK // 2 - 1)
                    def _():
                        pltpu.async_copy(h_hbm.at[row_v.at[par, p + 1]],
                                         gbufs[0], gsems[0])

                    @pl.when((k == BLK // 2 - 1) & (bb < NBLK - 1))
                    def _():
                        pltpu.make_async_copy(
                            row_hbm.at[pl.ds(base + BLK, BLK)],
                            row_v.at[1 - par], stg).wait()
                        pltpu.async_copy(h_hbm.at[row_v.at[1 - par, 0]],
                                         gbufs[0], gsems[0])

                @pl.when((bb > 0) | (k > 0))
                def _():
                    pltpu.make_async_copy(
                        sbufs[u], acc_sh.at[col_v.at[par, p]],
                        ssems[u]).wait()

                scale_chunk(gbufs[u], sbufs[u], p)
                pltpu.async_copy(sbufs[u], acc_sh.at[col_v.at[par, p]],
                                 ssems[u], add=True)
            return carry1

        lax.fori_loop(0, BLK // 2, duo, 0)
        return carry

    lax.fori_loop(0, NBLK, block, 0)

    for u in range(2):
        pltpu.make_async_copy(sbufs[u], acc_sh.at[col_v.at[0, 0]],
                              ssems[u]).wait()

    plsc.subcore_barrier()
    pltpu.sync_copy(
        acc_sh.at[pl.ds(s * RPT, RPT)],
        accp_hbm.at[c, pl.ds(s * RPT, RPT)],
    )


_msg_call = pl.kernel(
    _msg_body,
    out_type=jax.ShapeDtypeStruct((NC, NPAD, D), jnp.float32),
    mesh=_sc_mesh,
    compiler_params=_sc_params,
    scratch_types=[
        pltpu.VMEM((2, BLK, MC), jnp.int32),
        pltpu.VMEM((2, BLK, MC), jnp.int32),
        pltpu.VMEM((BLK, MC), jnp.float32),
        pltpu.VMEM((MC, D), jnp.float32),
        pltpu.VMEM((MC, D), jnp.float32),
        pltpu.VMEM((MC, D), jnp.float32),
        pltpu.VMEM((MC, D), jnp.float32),
        pltpu.VMEM_SHARED((NPAD, D), jnp.float32),
        pltpu.SemaphoreType.DMA,
        pltpu.SemaphoreType.DMA,
        pltpu.SemaphoreType.DMA,
        pltpu.SemaphoreType.DMA,
        pltpu.SemaphoreType.DMA,
    ],
)


MM_BLK = 1280


def _mm_body(x_ref, w_ref, o_ref):
    o_ref[...] = jnp.dot(x_ref[...], w_ref[...],
                         preferred_element_type=jnp.float32)


_mm_call = pl.pallas_call(
    _mm_body,
    grid=(NPAD // MM_BLK,),
    in_specs=[
        pl.BlockSpec((MM_BLK, D), lambda i: (i, 0)),
        pl.BlockSpec((D, D), lambda i: (0, 0)),
    ],
    out_specs=pl.BlockSpec((MM_BLK, D), lambda i: (i, 0)),
    out_shape=jax.ShapeDtypeStruct((NPAD, D), jnp.float32),
)


def _dis_body(degp_ref, dis_ref):
    deg = degp_ref[0, :] + degp_ref[1, :]
    safe = jnp.where(deg > 0, deg, 1.0)
    dis_ref[...] = jnp.where(deg > 0, lax.rsqrt(safe), 0.0)


_dis_call = pl.pallas_call(
    _dis_body,
    out_shape=jax.ShapeDtypeStruct((NPAD,), jnp.float32),
)


FIN_BLK = 1280


def _fin_body(accp_ref, dis_ref, b_ref, o_ref):
    acc = accp_ref[0] + accp_ref[1]
    o = acc * dis_ref[...] + b_ref[...]
    m = jnp.max(o, axis=1, keepdims=True)
    e = jnp.exp(o - m)
    o_ref[...] = e / jnp.sum(e, axis=1, keepdims=True)


_fin_call = pl.pallas_call(
    _fin_body,
    grid=(NPAD // FIN_BLK,),
    in_specs=[
        pl.BlockSpec((NC, FIN_BLK, D), lambda i: (0, i, 0)),
        pl.BlockSpec((FIN_BLK, 1), lambda i: (i, 0)),
        pl.BlockSpec((1, D), lambda i: (0, 0)),
    ],
    out_specs=pl.BlockSpec((FIN_BLK, D), lambda i: (i, 0)),
    out_shape=jax.ShapeDtypeStruct((NPAD, D), jnp.float32),
)


def kernel(x, edge_index, edge_weight, W, b):
    row = edge_index[0].astype(jnp.int32)
    col = edge_index[1].astype(jnp.int32)
    pad = EP - E
    rowp = jnp.concatenate([row, jnp.full((pad,), PAD_IDX, jnp.int32)])
    colp = jnp.concatenate([col, jnp.full((pad,), PAD_IDX, jnp.int32)])
    ewp = jnp.concatenate(
        [edge_weight.astype(jnp.float32), jnp.zeros((pad,), jnp.float32)])
    xp = jnp.concatenate(
        [x.astype(jnp.float32), jnp.zeros((NPAD - N, D), jnp.float32)])

    degp = _deg_call(colp.reshape(EP // CHUNK, CHUNK),
                     ewp.reshape(EP // CHUNK, CHUNK))
    h = _mm_call(xp, W.astype(jnp.float32))
    dis = _dis_call(degp)
    row64 = rowp.reshape(EP // MC, MC)
    gw = _gw_call(row64, ewp.reshape(EP // MC, MC), dis)
    accp = _msg_call(row64, colp.reshape(EP // MC, MC), gw, h)
    out = _fin_call(accp, dis.reshape(NPAD, 1),
                    b.astype(jnp.float32).reshape(1, D))
    return out[:N]


# speedup vs baseline: 1.1238x; 1.0363x over previous
import functools

import jax
import jax.numpy as jnp
import numpy as np
from jax import lax
from jax.experimental import pallas as pl
from jax.experimental.pallas import tpu as pltpu
from jax.experimental.pallas import tpu_sc as plsc

N = 10000
E = 320000
D = 128

NC = 2
NS = 16
LANES = 16
NW = NC * NS

CHUNK = 128
CPT = 80
EP = NW * CPT * CHUNK
NPAD = 10240
RPT = NPAD // NS
PAD_IDX = N + 16

MC = 64
MCPT = EP // (NW * MC)
BLK = 8
NBLK = MCPT // BLK

_sc_mesh = plsc.VectorSubcoreMesh(core_axis_name="c", subcore_axis_name="s")


def _deg_body(col_hbm, ew_hbm, degp_hbm, col_v, ew_v, zb_v, acc_sh):
    c = lax.axis_index("c")
    s = lax.axis_index("s")
    w = c * NS + s

    pltpu.sync_copy(col_hbm.at[pl.ds(w * CPT, CPT)], col_v)
    pltpu.sync_copy(ew_hbm.at[pl.ds(w * CPT, CPT)], ew_v)

    def zero(i, carry):
        zb_v[pl.ds(i * LANES, LANES)] = jnp.zeros((LANES,), jnp.float32)
        return carry

    lax.fori_loop(0, RPT // LANES, zero, 0)
    pltpu.sync_copy(zb_v, acc_sh.at[pl.ds(s * RPT, RPT)])
    plsc.subcore_barrier()

    def body(j, carry):
        pltpu.sync_copy(ew_v.at[j], acc_sh.at[col_v.at[j]], add=True)
        return carry

    lax.fori_loop(0, CPT, body, 0)
    plsc.subcore_barrier()
    pltpu.sync_copy(
        acc_sh.at[pl.ds(s * RPT, RPT)],
        degp_hbm.at[c, pl.ds(s * RPT, RPT)],
    )


_sc_params = pltpu.CompilerParams(needs_layout_passes=False)

_deg_call = pl.kernel(
    _deg_body,
    out_type=jax.ShapeDtypeStruct((NC, NPAD), jnp.float32),
    mesh=_sc_mesh,
    compiler_params=_sc_params,
    scratch_types=[
        pltpu.VMEM((CPT, CHUNK), jnp.int32),
        pltpu.VMEM((CPT, CHUNK), jnp.float32),
        pltpu.VMEM((RPT,), jnp.float32),
        pltpu.VMEM_SHARED((NPAD,), jnp.float32),
    ],
)


def _msg_body(row_hbm, col_hbm, gw_hbm, h_hbm, accp_hbm,
              row_v, col_v, gw_v, ga, gb, sa, sb, acc_sh,
              g0, g1, s0, s1, stg):
    c = lax.axis_index("c")
    s = lax.axis_index("s")
    w = c * NS + s
    tbase = w * MCPT

    gbufs = [ga, gb]
    sbufs = [sa, sb]
    gsems = [g0, g1]
    ssems = [s0, s1]

    def zrow(i, carry):
        for v in range(D // LANES):
            sa[i, pl.ds(v * LANES, LANES)] = jnp.zeros(
                (LANES,), jnp.float32)
        return carry

    lax.fori_loop(0, MC, zrow, 0)

    def zcopy(kz, carry):
        pltpu.sync_copy(sa, acc_sh.at[pl.ds(s * RPT + kz * MC, MC)])
        return carry

    lax.fori_loop(0, RPT // MC, zcopy, 0)
    plsc.subcore_barrier()

    def scale_chunk(gbuf, sbuf, p):
        def grp(gi, carry):
            g16 = gw_v[p, pl.ds(gi * LANES, LANES)]
            for j in range(LANES):
                e = gi * LANES + j
                g = jnp.broadcast_to(g16[j], (LANES,))
                for v in range(D // LANES):
                    sl = pl.ds(v * LANES, LANES)
                    sbuf[e, sl] = gbuf[e, sl] * g
            return carry

        lax.fori_loop(0, MC // LANES, grp, 0)

    pltpu.sync_copy(row_hbm.at[pl.ds(tbase, BLK)], row_v.at[0])
    pltpu.sync_copy(col_hbm.at[pl.ds(tbase, BLK)], col_v.at[0])
    pltpu.sync_copy(gw_hbm.at[pl.ds(tbase, BLK)], gw_v)
    pltpu.async_copy(h_hbm.at[row_v.at[0, 0]], ga, g0)

    def block(bb, carry):
        par = bb & 1
        base = tbase + bb * BLK

        @pl.when(bb > 0)
        def _():
            pltpu.sync_copy(col_hbm.at[pl.ds(base, BLK)], col_v.at[par])
            pltpu.sync_copy(gw_hbm.at[pl.ds(base, BLK)], gw_v)

        @pl.when(bb < NBLK - 1)
        def _():
            pltpu.async_copy(row_hbm.at[pl.ds(base + BLK, BLK)],
                             row_v.at[1 - par], stg)

        def duo(k, carry1):
            for u in range(2):
                p = 2 * k + u
                pltpu.make_async_copy(h_hbm.at[row_v.at[par, p]],
                                      gbufs[u], gsems[u]).wait()
                if u == 0:
                    pltpu.async_copy(h_hbm.at[row_v.at[par, p + 1]],
                                     gbufs[1], gsems[1])
                else:
                    @pl.when(k < BLK // 2 - 1)
                    def _():
                        pltpu.async_copy(h_hbm.at[row_v.at[par, p + 1]],
                                         gbufs[0], gsems[0])

                    @pl.when((k == BLK // 2 - 1) & (bb < NBLK - 1))
                    def _():
                        pltpu.make_async_copy(
                            row_hbm.at[pl.ds(base + BLK, BLK)],
                            row_v.at[1 - par], stg).wait()
                        pltpu.async_copy(h_hbm.at[row_v.at[1 - par, 0]],
                                         gbufs[0], gsems[0])

                @pl.when((bb > 0) | (k > 0))
                def _():
                    pltpu.make_async_copy(
                        sbufs[u], acc_sh.at[col_v.at[par, p]],
                        ssems[u]).wait()

                scale_chunk(gbufs[u], sbufs[u], p)
                pltpu.async_copy(sbufs[u], acc_sh.at[col_v.at[par, p]],
                                 ssems[u], add=True)
            return carry1

        lax.fori_loop(0, BLK // 2, duo, 0)
        return carry

    lax.fori_loop(0, NBLK, block, 0)

    for u in range(2):
        pltpu.make_async_copy(sbufs[u], acc_sh.at[col_v.at[0, 0]],
                              ssems[u]).wait()

    plsc.subcore_barrier()
    pltpu.sync_copy(
        acc_sh.at[pl.ds(s * RPT, RPT)],
        accp_hbm.at[c, pl.ds(s * RPT, RPT)],
    )


_msg_call = pl.kernel(
    _msg_body,
    out_type=jax.ShapeDtypeStruct((NC, NPAD, D), jnp.float32),
    mesh=_sc_mesh,
    compiler_params=_sc_params,
    scratch_types=[
        pltpu.VMEM((2, BLK, MC), jnp.int32),
        pltpu.VMEM((2, BLK, MC), jnp.int32),
        pltpu.VMEM((BLK, MC), jnp.float32),
        pltpu.VMEM((MC, D), jnp.float32),
        pltpu.VMEM((MC, D), jnp.float32),
        pltpu.VMEM((MC, D), jnp.float32),
        pltpu.VMEM((MC, D), jnp.float32),
        pltpu.VMEM_SHARED((NPAD, D), jnp.float32),
        pltpu.SemaphoreType.DMA,
        pltpu.SemaphoreType.DMA,
        pltpu.SemaphoreType.DMA,
        pltpu.SemaphoreType.DMA,
        pltpu.SemaphoreType.DMA,
    ],
)


MM_BLK = 1280


def _mm_body(x_ref, w_ref, dis_ref, o_ref):
    o_ref[...] = jnp.dot(x_ref[...], w_ref[...],
                         preferred_element_type=jnp.float32) * dis_ref[...]


_mm_call = pl.pallas_call(
    _mm_body,
    grid=(NPAD // MM_BLK,),
    in_specs=[
        pl.BlockSpec((MM_BLK, D), lambda i: (i, 0)),
        pl.BlockSpec((D, D), lambda i: (0, 0)),
        pl.BlockSpec((MM_BLK, 1), lambda i: (i, 0)),
    ],
    out_specs=pl.BlockSpec((MM_BLK, D), lambda i: (i, 0)),
    out_shape=jax.ShapeDtypeStruct((NPAD, D), jnp.float32),
)


def _dis_body(degp_ref, dis_ref):
    deg = degp_ref[0, :] + degp_ref[1, :]
    safe = jnp.where(deg > 0, deg, 1.0)
    dis_ref[...] = jnp.where(deg > 0, lax.rsqrt(safe), 0.0)


_dis_call = pl.pallas_call(
    _dis_body,
    out_shape=jax.ShapeDtypeStruct((NPAD,), jnp.float32),
)


FIN_BLK = 1280


def _fin_body(accp_ref, dis_ref, b_ref, o_ref):
    acc = accp_ref[0] + accp_ref[1]
    o = acc * dis_ref[...] + b_ref[...]
    m = jnp.max(o, axis=1, keepdims=True)
    e = jnp.exp(o - m)
    o_ref[...] = e / jnp.sum(e, axis=1, keepdims=True)


_fin_call = pl.pallas_call(
    _fin_body,
    grid=(NPAD // FIN_BLK,),
    in_specs=[
        pl.BlockSpec((NC, FIN_BLK, D), lambda i: (0, i, 0)),
        pl.BlockSpec((FIN_BLK, 1), lambda i: (i, 0)),
        pl.BlockSpec((1, D), lambda i: (0, 0)),
    ],
    out_specs=pl.BlockSpec((FIN_BLK, D), lambda i: (i, 0)),
    out_shape=jax.ShapeDtypeStruct((NPAD, D), jnp.float32),
)


def kernel(x, edge_index, edge_weight, W, b):
    row = edge_index[0].astype(jnp.int32)
    col = edge_index[1].astype(jnp.int32)
    pad = EP - E
    rowp = jnp.concatenate([row, jnp.full((pad,), PAD_IDX, jnp.int32)])
    colp = jnp.concatenate([col, jnp.full((pad,), PAD_IDX, jnp.int32)])
    ewp = jnp.concatenate(
        [edge_weight.astype(jnp.float32), jnp.zeros((pad,), jnp.float32)])
    xp = jnp.concatenate(
        [x.astype(jnp.float32), jnp.zeros((NPAD - N, D), jnp.float32)])

    degp = _deg_call(colp.reshape(EP // CHUNK, CHUNK),
                     ewp.reshape(EP // CHUNK, CHUNK))
    dis = _dis_call(degp)
    h = _mm_call(xp, W.astype(jnp.float32), dis.reshape(NPAD, 1))
    row64 = rowp.reshape(EP // MC, MC)
    accp = _msg_call(row64, colp.reshape(EP // MC, MC),
                     ewp.reshape(EP // MC, MC), h)
    out = _fin_call(accp, dis.reshape(NPAD, 1),
                    b.astype(jnp.float32).reshape(1, D))
    return out[:N]


# speedup vs baseline: 1.2125x; 1.0790x over previous
import functools

import jax
import jax.numpy as jnp
import numpy as np
from jax import lax
from jax.experimental import pallas as pl
from jax.experimental.pallas import tpu as pltpu
from jax.experimental.pallas import tpu_sc as plsc

N = 10000
E = 320000
D = 128

NC = 2
NS = 16
LANES = 16
NW = NC * NS

CHUNK = 128
CPT = 80
EP = NW * CPT * CHUNK
NPAD = 10240
RPT = NPAD // NS
PAD_IDX = N + 16

MC = 64
MCPT = EP // (NW * MC)
BLK = 8
NBLK = MCPT // BLK

_sc_mesh = plsc.VectorSubcoreMesh(core_axis_name="c", subcore_axis_name="s")


def _deg_body(col_hbm, ew_hbm, degp_hbm, col_v, ew_v, zb_v, acc_sh):
    c = lax.axis_index("c")
    s = lax.axis_index("s")
    w = c * NS + s

    pltpu.sync_copy(col_hbm.at[pl.ds(w * CPT, CPT)], col_v)
    pltpu.sync_copy(ew_hbm.at[pl.ds(w * CPT, CPT)], ew_v)

    def zero(i, carry):
        zb_v[pl.ds(i * LANES, LANES)] = jnp.zeros((LANES,), jnp.float32)
        return carry

    lax.fori_loop(0, RPT // LANES, zero, 0)
    pltpu.sync_copy(zb_v, acc_sh.at[pl.ds(s * RPT, RPT)])
    plsc.subcore_barrier()

    def body(j, carry):
        pltpu.sync_copy(ew_v.at[j], acc_sh.at[col_v.at[j]], add=True)
        return carry

    lax.fori_loop(0, CPT, body, 0)
    plsc.subcore_barrier()
    pltpu.sync_copy(
        acc_sh.at[pl.ds(s * RPT, RPT)],
        degp_hbm.at[c, pl.ds(s * RPT, RPT)],
    )


_sc_params = pltpu.CompilerParams(needs_layout_passes=False)

_deg_call = pl.kernel(
    _deg_body,
    out_type=jax.ShapeDtypeStruct((NC, NPAD), jnp.float32),
    mesh=_sc_mesh,
    compiler_params=_sc_params,
    scratch_types=[
        pltpu.VMEM((CPT, CHUNK), jnp.int32),
        pltpu.VMEM((CPT, CHUNK), jnp.float32),
        pltpu.VMEM((RPT,), jnp.float32),
        pltpu.VMEM_SHARED((NPAD,), jnp.float32),
    ],
)


def _msg_body(row_hbm, col_hbm, gw_hbm, h_hbm, accp_hbm,
              row_v, col_v, gw_v, ga, gb, sa, sb, acc_sh,
              g0, g1, s0, s1, stg):
    c = lax.axis_index("c")
    s = lax.axis_index("s")
    w = c * NS + s
    tbase = w * MCPT

    gbufs = [ga, gb]
    sbufs = [sa, sb]
    gsems = [g0, g1]
    ssems = [s0, s1]

    def zrow(i, carry):
        for v in range(D // LANES):
            sa[i, pl.ds(v * LANES, LANES)] = jnp.zeros(
                (LANES,), jnp.float32)
        return carry

    lax.fori_loop(0, MC, zrow, 0)

    def zcopy(kz, carry):
        pltpu.sync_copy(sa, acc_sh.at[pl.ds(s * RPT + kz * MC, MC)])
        return carry

    lax.fori_loop(0, RPT // MC, zcopy, 0)
    plsc.subcore_barrier()

    def scale_chunk(gbuf, sbuf, p):
        def grp(gi, carry):
            g16 = gw_v[p, pl.ds(gi * LANES, LANES)]
            for j in range(LANES):
                e = gi * LANES + j
                g = jnp.broadcast_to(g16[j], (LANES,))
                for v in range(D // LANES):
                    sl = pl.ds(v * LANES, LANES)
                    sbuf[e, sl] = gbuf[e, sl] * g
            return carry

        lax.fori_loop(0, MC // LANES, grp, 0)

    pltpu.sync_copy(row_hbm.at[pl.ds(tbase, BLK)], row_v.at[0])
    pltpu.sync_copy(col_hbm.at[pl.ds(tbase, BLK)], col_v.at[0])
    pltpu.sync_copy(gw_hbm.at[pl.ds(tbase, BLK)], gw_v)
    pltpu.async_copy(h_hbm.at[row_v.at[0, 0]], ga, g0)

    def block(bb, carry):
        par = bb & 1
        base = tbase + bb * BLK

        @pl.when(bb > 0)
        def _():
            pltpu.sync_copy(col_hbm.at[pl.ds(base, BLK)], col_v.at[par])
            pltpu.sync_copy(gw_hbm.at[pl.ds(base, BLK)], gw_v)

        @pl.when(bb < NBLK - 1)
        def _():
            pltpu.async_copy(row_hbm.at[pl.ds(base + BLK, BLK)],
                             row_v.at[1 - par], stg)

        def duo(k, carry1):
            for u in range(2):
                p = 2 * k + u
                if u == 0:
                    pltpu.async_copy(h_hbm.at[row_v.at[par, p + 1]],
                                     gbufs[1], gsems[1])
                else:
                    @pl.when(k < BLK // 2 - 1)
                    def _():
                        pltpu.async_copy(h_hbm.at[row_v.at[par, p + 1]],
                                         gbufs[0], gsems[0])

                    @pl.when((k == BLK // 2 - 1) & (bb < NBLK - 1))
                    def _():
                        pltpu.make_async_copy(
                            row_hbm.at[pl.ds(base + BLK, BLK)],
                            row_v.at[1 - par], stg).wait()
                        pltpu.async_copy(h_hbm.at[row_v.at[1 - par, 0]],
                                         gbufs[0], gsems[0])

                pltpu.make_async_copy(h_hbm.at[row_v.at[par, p]],
                                      gbufs[u], gsems[u]).wait()

                @pl.when((bb > 0) | (k > 0))
                def _():
                    pltpu.make_async_copy(
                        sbufs[u], acc_sh.at[col_v.at[par, p]],
                        ssems[u]).wait()

                scale_chunk(gbufs[u], sbufs[u], p)
                pltpu.async_copy(sbufs[u], acc_sh.at[col_v.at[par, p]],
                                 ssems[u], add=True)
            return carry1

        lax.fori_loop(0, BLK // 2, duo, 0)
        return carry

    lax.fori_loop(0, NBLK, block, 0)

    for u in range(2):
        pltpu.make_async_copy(sbufs[u], acc_sh.at[col_v.at[0, 0]],
                              ssems[u]).wait()

    plsc.subcore_barrier()
    pltpu.sync_copy(
        acc_sh.at[pl.ds(s * RPT, RPT)],
        accp_hbm.at[c, pl.ds(s * RPT, RPT)],
    )


_msg_call = pl.kernel(
    _msg_body,
    out_type=jax.ShapeDtypeStruct((NC, NPAD, D), jnp.float32),
    mesh=_sc_mesh,
    compiler_params=_sc_params,
    scratch_types=[
        pltpu.VMEM((2, BLK, MC), jnp.int32),
        pltpu.VMEM((2, BLK, MC), jnp.int32),
        pltpu.VMEM((BLK, MC), jnp.float32),
        pltpu.VMEM((MC, D), jnp.float32),
        pltpu.VMEM((MC, D), jnp.float32),
        pltpu.VMEM((MC, D), jnp.float32),
        pltpu.VMEM((MC, D), jnp.float32),
        pltpu.VMEM_SHARED((NPAD, D), jnp.float32),
        pltpu.SemaphoreType.DMA,
        pltpu.SemaphoreType.DMA,
        pltpu.SemaphoreType.DMA,
        pltpu.SemaphoreType.DMA,
        pltpu.SemaphoreType.DMA,
    ],
)


MM_BLK = 1280


def _mm_body(x_ref, w_ref, dis_ref, o_ref):
    o_ref[...] = jnp.dot(x_ref[...], w_ref[...],
                         preferred_element_type=jnp.float32) * dis_ref[...]


_mm_call = pl.pallas_call(
    _mm_body,
    grid=(NPAD // MM_BLK,),
    in_specs=[
        pl.BlockSpec((MM_BLK, D), lambda i: (i, 0)),
        pl.BlockSpec((D, D), lambda i: (0, 0)),
        pl.BlockSpec((MM_BLK, 1), lambda i: (i, 0)),
    ],
    out_specs=pl.BlockSpec((MM_BLK, D), lambda i: (i, 0)),
    out_shape=jax.ShapeDtypeStruct((NPAD, D), jnp.float32),
)


def _dis_body(degp_ref, dis_ref):
    deg = degp_ref[0, :] + degp_ref[1, :]
    safe = jnp.where(deg > 0, deg, 1.0)
    dis_ref[...] = jnp.where(deg > 0, lax.rsqrt(safe), 0.0)


_dis_call = pl.pallas_call(
    _dis_body,
    out_shape=jax.ShapeDtypeStruct((NPAD,), jnp.float32),
)


FIN_BLK = 1280


def _fin_body(accp_ref, dis_ref, b_ref, o_ref):
    acc = accp_ref[0] + accp_ref[1]
    o = acc * dis_ref[...] + b_ref[...]
    m = jnp.max(o, axis=1, keepdims=True)
    e = jnp.exp(o - m)
    o_ref[...] = e / jnp.sum(e, axis=1, keepdims=True)


_fin_call = pl.pallas_call(
    _fin_body,
    grid=(NPAD // FIN_BLK,),
    in_specs=[
        pl.BlockSpec((NC, FIN_BLK, D), lambda i: (0, i, 0)),
        pl.BlockSpec((FIN_BLK, 1), lambda i: (i, 0)),
        pl.BlockSpec((1, D), lambda i: (0, 0)),
    ],
    out_specs=pl.BlockSpec((FIN_BLK, D), lambda i: (i, 0)),
    out_shape=jax.ShapeDtypeStruct((NPAD, D), jnp.float32),
)


def kernel(x, edge_index, edge_weight, W, b):
    row = edge_index[0].astype(jnp.int32)
    col = edge_index[1].astype(jnp.int32)
    pad = EP - E
    rowp = jnp.concatenate([row, jnp.full((pad,), PAD_IDX, jnp.int32)])
    colp = jnp.concatenate([col, jnp.full((pad,), PAD_IDX, jnp.int32)])
    ewp = jnp.concatenate(
        [edge_weight.astype(jnp.float32), jnp.zeros((pad,), jnp.float32)])
    xp = jnp.concatenate(
        [x.astype(jnp.float32), jnp.zeros((NPAD - N, D), jnp.float32)])

    degp = _deg_call(colp.reshape(EP // CHUNK, CHUNK),
                     ewp.reshape(EP // CHUNK, CHUNK))
    dis = _dis_call(degp)
    h = _mm_call(xp, W.astype(jnp.float32), dis.reshape(NPAD, 1))
    row64 = rowp.reshape(EP // MC, MC)
    accp = _msg_call(row64, colp.reshape(EP // MC, MC),
                     ewp.reshape(EP // MC, MC), h)
    out = _fin_call(accp, dis.reshape(NPAD, 1),
                    b.astype(jnp.float32).reshape(1, D))
    return out[:N]


# speedup vs baseline: 1.2165x; 1.0032x over previous
import functools

import jax
import jax.numpy as jnp
import numpy as np
from jax import lax
from jax.experimental import pallas as pl
from jax.experimental.pallas import tpu as pltpu
from jax.experimental.pallas import tpu_sc as plsc

N = 10000
E = 320000
D = 128

NC = 2
NS = 16
LANES = 16
NW = NC * NS

CHUNK = 128
CPT = 80
EP = NW * CPT * CHUNK
NPAD = 10240
RPT = NPAD // NS
PAD_IDX = N + 16

MC = 64
MCPT = EP // (NW * MC)
BLK = 8
NBLK = MCPT // BLK

_sc_mesh = plsc.VectorSubcoreMesh(core_axis_name="c", subcore_axis_name="s")


def _deg_body(col_hbm, ew_hbm, degp_hbm, col_v, ew_v, zb_v, acc_sh):
    c = lax.axis_index("c")
    s = lax.axis_index("s")
    w = c * NS + s

    pltpu.sync_copy(col_hbm.at[pl.ds(w * CPT, CPT)], col_v)
    pltpu.sync_copy(ew_hbm.at[pl.ds(w * CPT, CPT)], ew_v)

    def zero(i, carry):
        zb_v[pl.ds(i * LANES, LANES)] = jnp.zeros((LANES,), jnp.float32)
        return carry

    lax.fori_loop(0, RPT // LANES, zero, 0)
    pltpu.sync_copy(zb_v, acc_sh.at[pl.ds(s * RPT, RPT)])
    plsc.subcore_barrier()

    def body(j, carry):
        pltpu.sync_copy(ew_v.at[j], acc_sh.at[col_v.at[j]], add=True)
        return carry

    lax.fori_loop(0, CPT, body, 0)
    plsc.subcore_barrier()
    pltpu.sync_copy(
        acc_sh.at[pl.ds(s * RPT, RPT)],
        degp_hbm.at[c, pl.ds(s * RPT, RPT)],
    )


_sc_params = pltpu.CompilerParams(needs_layout_passes=False)

_deg_call = pl.kernel(
    _deg_body,
    out_type=jax.ShapeDtypeStruct((NC, NPAD), jnp.float32),
    mesh=_sc_mesh,
    compiler_params=_sc_params,
    scratch_types=[
        pltpu.VMEM((CPT, CHUNK), jnp.int32),
        pltpu.VMEM((CPT, CHUNK), jnp.float32),
        pltpu.VMEM((RPT,), jnp.float32),
        pltpu.VMEM_SHARED((NPAD,), jnp.float32),
    ],
)


def _msg_body(row_hbm, col_hbm, gw_hbm, h_hbm, accp_hbm,
              row_v, col_v, gw_v, b0, b1, b2, b3, acc_sh,
              g0, g1, g2, g3, s0, s1, s2, s3, stg):
    c = lax.axis_index("c")
    s = lax.axis_index("s")
    w = c * NS + s
    tbase = w * MCPT

    bufs = [b0, b1, b2, b3]
    gsems = [g0, g1, g2, g3]
    ssems = [s0, s1, s2, s3]

    def zrow(i, carry):
        for v in range(D // LANES):
            b0[i, pl.ds(v * LANES, LANES)] = jnp.zeros(
                (LANES,), jnp.float32)
        return carry

    lax.fori_loop(0, MC, zrow, 0)

    def zcopy(kz, carry):
        pltpu.sync_copy(b0, acc_sh.at[pl.ds(s * RPT + kz * MC, MC)])
        return carry

    lax.fori_loop(0, RPT // MC, zcopy, 0)
    plsc.subcore_barrier()

    def scale_chunk(buf, p):
        def grp(gi, carry):
            g16 = gw_v[p, pl.ds(gi * LANES, LANES)]
            for j in range(LANES):
                e = gi * LANES + j
                g = jnp.broadcast_to(g16[j], (LANES,))
                for v in range(D // LANES):
                    sl = pl.ds(v * LANES, LANES)
                    buf[e, sl] = buf[e, sl] * g
            return carry

        lax.fori_loop(0, MC // LANES, grp, 0)

    pltpu.sync_copy(row_hbm.at[pl.ds(tbase, BLK)], row_v.at[0])
    pltpu.sync_copy(col_hbm.at[pl.ds(tbase, BLK)], col_v.at[0])
    pltpu.sync_copy(gw_hbm.at[pl.ds(tbase, BLK)], gw_v)
    for q in range(3):
        pltpu.async_copy(h_hbm.at[row_v.at[0, q]], bufs[q], gsems[q])

    def block(bb, carry):
        par = bb & 1
        base = tbase + bb * BLK

        @pl.when(bb > 0)
        def _():
            pltpu.sync_copy(col_hbm.at[pl.ds(base, BLK)], col_v.at[par])
            pltpu.sync_copy(gw_hbm.at[pl.ds(base, BLK)], gw_v)

        @pl.when(bb < NBLK - 1)
        def _():
            pltpu.async_copy(row_hbm.at[pl.ds(base + BLK, BLK)],
                             row_v.at[1 - par], stg)

        for p in range(BLK):
            b = p % 4
            nb = (p + 3) % 4
            np_ = p + 3
            if p == 0:
                @pl.when(bb > 0)
                def _():
                    pltpu.make_async_copy(
                        bufs[nb], acc_sh.at[col_v.at[par, 0]],
                        ssems[nb]).wait()
            else:
                pltpu.make_async_copy(
                    bufs[nb], acc_sh.at[col_v.at[par, 0]],
                    ssems[nb]).wait()
            if np_ < BLK:
                pltpu.async_copy(h_hbm.at[row_v.at[par, np_]],
                                 bufs[nb], gsems[nb])
            else:
                if np_ == BLK:
                    @pl.when(bb < NBLK - 1)
                    def _():
                        pltpu.make_async_copy(
                            row_hbm.at[pl.ds(base + BLK, BLK)],
                            row_v.at[1 - par], stg).wait()

                @pl.when(bb < NBLK - 1)
                def _():
                    pltpu.async_copy(h_hbm.at[row_v.at[1 - par, np_ - BLK]],
                                     bufs[nb], gsems[nb])

            pltpu.make_async_copy(h_hbm.at[row_v.at[par, p]],
                                  bufs[b], gsems[b]).wait()
            scale_chunk(bufs[b], p)
            pltpu.async_copy(bufs[b], acc_sh.at[col_v.at[par, p]],
                             ssems[b], add=True)
        return carry

    lax.fori_loop(0, NBLK, block, 0)

    pltpu.make_async_copy(bufs[(BLK - 1) % 4], acc_sh.at[col_v.at[0, 0]],
                          ssems[(BLK - 1) % 4]).wait()

    plsc.subcore_barrier()
    pltpu.sync_copy(
        acc_sh.at[pl.ds(s * RPT, RPT)],
        accp_hbm.at[c, pl.ds(s * RPT, RPT)],
    )


_msg_call = pl.kernel(
    _msg_body,
    out_type=jax.ShapeDtypeStruct((NC, NPAD, D), jnp.float32),
    mesh=_sc_mesh,
    compiler_params=_sc_params,
    scratch_types=[
        pltpu.VMEM((2, BLK, MC), jnp.int32),
        pltpu.VMEM((2, BLK, MC), jnp.int32),
        pltpu.VMEM((BLK, MC), jnp.float32),
        pltpu.VMEM((MC, D), jnp.float32),
        pltpu.VMEM((MC, D), jnp.float32),
        pltpu.VMEM((MC, D), jnp.float32),
        pltpu.VMEM((MC, D), jnp.float32),
        pltpu.VMEM_SHARED((NPAD, D), jnp.float32),
        pltpu.SemaphoreType.DMA,
        pltpu.SemaphoreType.DMA,
        pltpu.SemaphoreType.DMA,
        pltpu.SemaphoreType.DMA,
        pltpu.SemaphoreType.DMA,
        pltpu.SemaphoreType.DMA,
        pltpu.SemaphoreType.DMA,
        pltpu.SemaphoreType.DMA,
        pltpu.SemaphoreType.DMA,
    ],
)


MM_BLK = 1280


def _mm_body(x_ref, w_ref, dis_ref, o_ref):
    o_ref[...] = jnp.dot(x_ref[...], w_ref[...],
                         preferred_element_type=jnp.float32) * dis_ref[...]


_mm_call = pl.pallas_call(
    _mm_body,
    grid=(NPAD // MM_BLK,),
    in_specs=[
        pl.BlockSpec((MM_BLK, D), lambda i: (i, 0)),
        pl.BlockSpec((D, D), lambda i: (0, 0)),
        pl.BlockSpec((MM_BLK, 1), lambda i: (i, 0)),
    ],
    out_specs=pl.BlockSpec((MM_BLK, D), lambda i: (i, 0)),
    out_shape=jax.ShapeDtypeStruct((NPAD, D), jnp.float32),
)


def _dis_body(degp_ref, dis_ref):
    deg = degp_ref[0, :] + degp_ref[1, :]
    safe = jnp.where(deg > 0, deg, 1.0)
    dis_ref[...] = jnp.where(deg > 0, lax.rsqrt(safe), 0.0)


_dis_call = pl.pallas_call(
    _dis_body,
    out_shape=jax.ShapeDtypeStruct((NPAD,), jnp.float32),
)


FIN_BLK = 1280


def _fin_body(accp_ref, dis_ref, b_ref, o_ref):
    acc = accp_ref[0] + accp_ref[1]
    o = acc * dis_ref[...] + b_ref[...]
    m = jnp.max(o, axis=1, keepdims=True)
    e = jnp.exp(o - m)
    o_ref[...] = e / jnp.sum(e, axis=1, keepdims=True)


_fin_call = pl.pallas_call(
    _fin_body,
    grid=(NPAD // FIN_BLK,),
    in_specs=[
        pl.BlockSpec((NC, FIN_BLK, D), lambda i: (0, i, 0)),
        pl.BlockSpec((FIN_BLK, 1), lambda i: (i, 0)),
        pl.BlockSpec((1, D), lambda i: (0, 0)),
    ],
    out_specs=pl.BlockSpec((FIN_BLK, D), lambda i: (i, 0)),
    out_shape=jax.ShapeDtypeStruct((NPAD, D), jnp.float32),
)


def kernel(x, edge_index, edge_weight, W, b):
    row = edge_index[0].astype(jnp.int32)
    col = edge_index[1].astype(jnp.int32)
    pad = EP - E
    rowp = jnp.concatenate([row, jnp.full((pad,), PAD_IDX, jnp.int32)])
    colp = jnp.concatenate([col, jnp.full((pad,), PAD_IDX, jnp.int32)])
    ewp = jnp.concatenate(
        [edge_weight.astype(jnp.float32), jnp.zeros((pad,), jnp.float32)])
    xp = jnp.concatenate(
        [x.astype(jnp.float32), jnp.zeros((NPAD - N, D), jnp.float32)])

    degp = _deg_call(colp.reshape(EP // CHUNK, CHUNK),
                     ewp.reshape(EP // CHUNK, CHUNK))
    dis = _dis_call(degp)
    h = _mm_call(xp, W.astype(jnp.float32), dis.reshape(NPAD, 1))
    row64 = rowp.reshape(EP // MC, MC)
    accp = _msg_call(row64, colp.reshape(EP // MC, MC),
                     ewp.reshape(EP // MC, MC), h)
    out = _fin_call(accp, dis.reshape(NPAD, 1),
                    b.astype(jnp.float32).reshape(1, D))
    return out[:N]
